# baseline (speedup 1.0000x reference)
import jax
import jax.numpy as jnp
from jax.experimental import pallas as pl
from jax.experimental.pallas import tpu as pltpu

_CHUNK = 2048
_NBUF = 4


def _router_stream(x_hbm, w_ref, b_ref, o_hbm, xbuf, obuf, in_sem, out_sem):
    n_tokens = x_hbm.shape[0]
    n_chunks = n_tokens // _CHUNK

    def in_copy(i, slot):
        return pltpu.make_async_copy(
            x_hbm.at[pl.ds(i * _CHUNK, _CHUNK), :], xbuf.at[slot],
            in_sem.at[slot])

    def out_copy(i, slot):
        return pltpu.make_async_copy(
            obuf.at[slot], o_hbm.at[pl.ds(i * _CHUNK, _CHUNK), :],
            out_sem.at[slot])

    for k in range(min(_NBUF, n_chunks)):
        in_copy(k, k).start()

    for i in range(n_chunks):
        slot = i % _NBUF
        in_copy(i, slot).wait()
        logits = jax.lax.dot_general(
            xbuf[slot], w_ref[...],
            dimension_numbers=(((1,), (1,)), ((), ())),
            preferred_element_type=jnp.float32,
        ) + b_ref[...]
        m = jnp.max(logits, axis=1, keepdims=True)
        e = jnp.exp(logits - m)
        if i >= _NBUF:
            out_copy(i - _NBUF, slot).wait()
        obuf[slot] = e / jnp.sum(e, axis=1, keepdims=True)
        out_copy(i, slot).start()
        nxt = i + _NBUF
        if nxt < n_chunks:
            in_copy(nxt, slot).start()

    for i in range(max(0, n_chunks - _NBUF), n_chunks):
        out_copy(i, i % _NBUF).wait()


@jax.jit
def kernel(x, W, b):
    n_tokens, d_model = x.shape
    n_experts = W.shape[0]
    b2 = b.reshape(1, n_experts)
    return pl.pallas_call(
        _router_stream,
        in_specs=[
            pl.BlockSpec(memory_space=pltpu.MemorySpace.HBM),
            pl.BlockSpec(memory_space=pltpu.MemorySpace.VMEM),
            pl.BlockSpec(memory_space=pltpu.MemorySpace.VMEM),
        ],
        out_specs=pl.BlockSpec(memory_space=pltpu.MemorySpace.HBM),
        out_shape=jax.ShapeDtypeStruct((n_tokens, n_experts), jnp.float32),
        scratch_shapes=[
            pltpu.VMEM((_NBUF, _CHUNK, d_model), jnp.float32),
            pltpu.VMEM((_NBUF, _CHUNK, n_experts), jnp.float32),
            pltpu.SemaphoreType.DMA((_NBUF,)),
            pltpu.SemaphoreType.DMA((_NBUF,)),
        ],
    )(x, W, b2)


# speedup vs baseline: 1.0465x; 1.0465x over previous
import jax
import jax.numpy as jnp
from jax.experimental import pallas as pl
from jax.experimental.pallas import tpu as pltpu

_CHUNK = 2048
_NBUF = 4


def _router_stream(x_hbm, w_ref, b_ref, o_ref, xbuf, in_sem):
    i = pl.program_id(0)

    def start_in(chunk, slot):
        pltpu.make_async_copy(
            x_hbm.at[pl.ds(chunk * _CHUNK, _CHUNK), :], xbuf.at[slot],
            in_sem.at[slot]).start()

    @pl.when(i == 0)
    def _prologue():
        for k in range(_NBUF):
            start_in(k, k)

    slot = jax.lax.rem(i, _NBUF)
    pltpu.make_async_copy(
        x_hbm.at[pl.ds(i * _CHUNK, _CHUNK), :], xbuf.at[slot],
        in_sem.at[slot]).wait()

    logits = jax.lax.dot_general(
        xbuf[slot], w_ref[...],
        dimension_numbers=(((1,), (1,)), ((), ())),
        preferred_element_type=jnp.float32,
    ) + b_ref[...]
    m = jnp.max(logits, axis=1, keepdims=True)
    e = jnp.exp(logits - m)
    o_ref[...] = e / jnp.sum(e, axis=1, keepdims=True)

    @pl.when(i + _NBUF < pl.num_programs(0))
    def _prefetch():
        start_in(i + _NBUF, slot)


@jax.jit
def kernel(x, W, b):
    n_tokens, d_model = x.shape
    n_experts = W.shape[0]
    b2 = b.reshape(1, n_experts)
    return pl.pallas_call(
        _router_stream,
        grid=(n_tokens // _CHUNK,),
        in_specs=[
            pl.BlockSpec(memory_space=pltpu.MemorySpace.HBM),
            pl.BlockSpec((n_experts, d_model), lambda i: (0, 0)),
            pl.BlockSpec((1, n_experts), lambda i: (0, 0)),
        ],
        out_specs=pl.BlockSpec((_CHUNK, n_experts), lambda i: (i, 0)),
        out_shape=jax.ShapeDtypeStruct((n_tokens, n_experts), jnp.float32),
        scratch_shapes=[
            pltpu.VMEM((_NBUF, _CHUNK, d_model), jnp.float32),
            pltpu.SemaphoreType.DMA((_NBUF,)),
        ],
        compiler_params=pltpu.CompilerParams(
            dimension_semantics=("arbitrary",),
        ),
    )(x, W, b2)
